# baseline (speedup 1.0000x reference)
import functools

import jax
import jax.numpy as jnp
from jax import lax
from jax.experimental import pallas as pl
from jax.experimental.pallas import tpu as pltpu
from jax.experimental.pallas import tpu_sc as plsc

_B, _S, _D = 1, 2048, 1024
_E, _K, _F, _R = 8, 2, 512, 8
_T = _B * _S
_TILE = 256
_NT = (_T * _K) // _TILE + _E
_NSLOT = _NT * _TILE
_CH = 256
_NC, _NS = 2, 16
_NW = _NC * _NS
_TPW = _T // _NW


def _dispatch_body(x16_ref, wr_ref, posa_ref, posb_ref, wpair_ref,
                   te_ref, nact_ref):
    logits = jax.lax.dot_general(
        x16_ref[...], wr_ref[...].astype(jnp.bfloat16),
        (((1,), (1,)), ((), ())),
        preferred_element_type=jnp.float32)
    m = jnp.max(logits, axis=-1, keepdims=True)
    p = jnp.exp(logits - m)
    p = p / jnp.sum(p, axis=-1, keepdims=True)
    lane = jax.lax.broadcasted_iota(jnp.int32, (_T, _E), 1)
    p1 = jnp.max(p, axis=-1, keepdims=True)
    i1 = jnp.min(jnp.where(p == p1, lane, _E), axis=-1, keepdims=True)
    m1 = lane == i1
    pr = jnp.where(m1, -1.0, p)
    p2 = jnp.max(pr, axis=-1, keepdims=True)
    i2 = jnp.min(jnp.where(pr == p2, lane, _E), axis=-1, keepdims=True)
    m2 = lane == i2
    denom = p1 + p2
    wpair_ref[:, 0:1] = p1 / denom
    wpair_ref[:, 1:2] = p2 / denom

    d = (m1 | m2).astype(jnp.bfloat16)
    rsub = jax.lax.broadcasted_iota(jnp.int32, (_CH, _CH), 0)
    csub = jax.lax.broadcasted_iota(jnp.int32, (_CH, _CH), 1)
    ls = (csub < rsub).astype(jnp.bfloat16)
    off = jnp.zeros((1, _E), jnp.float32)
    rparts = []
    for c in range(_T // _CH):
        dc = d[c * _CH:(c + 1) * _CH, :]
        rc = jax.lax.dot_general(ls, dc, (((1,), (0,)), ((), ())),
                                 preferred_element_type=jnp.float32)
        rparts.append(rc + off)
        off = off + jnp.sum(dc.astype(jnp.float32), axis=0, keepdims=True)
    rk = jnp.concatenate(rparts, axis=0)

    cnt = off
    rs = ((cnt.astype(jnp.int32) + (_TILE - 1)) // _TILE * _TILE)
    rs_f = rs.astype(jnp.float32)
    er = jax.lax.broadcasted_iota(jnp.int32, (_E, _E), 0)
    ec = jax.lax.broadcasted_iota(jnp.int32, (_E, _E), 1)
    umask = (er < ec).astype(jnp.float32)
    sp = jax.lax.dot_general(rs_f, umask, (((1,), (0,)), ((), ())),
                             preferred_element_type=jnp.float32,
                             precision=jax.lax.Precision.HIGHEST)

    spb = sp + rk
    posa_ref[...] = jnp.sum(jnp.where(m1, spb, 0.0), axis=-1,
                            keepdims=True).astype(jnp.int32)
    posb_ref[...] = jnp.sum(jnp.where(m2, spb, 0.0), axis=-1,
                            keepdims=True).astype(jnp.int32)

    trow = jax.lax.broadcasted_iota(jnp.int32, (128, _E), 0) * _TILE
    te_ref[...] = (jnp.sum((trow >= sp.astype(jnp.int32)).astype(jnp.int32),
                           axis=-1, keepdims=True) - 1)
    nact_ref[...] = jnp.sum(rs, axis=-1, keepdims=True) // _TILE


def _sc_scatter_body(x_hbm, posa_hbm, posb_hbm, xs_hbm,
                     idxa_v, idxb_v, rows_v, sem):
    wid = lax.axis_index("s") * _NC + lax.axis_index("c")
    base = wid * _TPW
    pltpu.sync_copy(posa_hbm.at[pl.ds(base, _TPW)], idxa_v)
    pltpu.sync_copy(posb_hbm.at[pl.ds(base, _TPW)], idxb_v)
    pltpu.sync_copy(x_hbm.at[pl.ds(base, _TPW)], rows_v)
    pltpu.async_copy(rows_v, xs_hbm.at[idxa_v], sem).wait()
    pltpu.async_copy(rows_v, xs_hbm.at[idxb_v], sem).wait()


def _grouped_body(te_ref, nact_ref, xs_ref, wg_ref, wu_ref, wd_ref,
                  ag_ref, bg_ref, au_ref, bu_ref, ad_ref, bd_ref,
                  ys_ref, wgu_s, wdm_s):
    j = pl.program_id(0)
    te_prev = te_ref[jnp.maximum(j - 1, 0)]

    @pl.when((j == 0) | (te_ref[j] != te_prev))
    def _merge():
        def merged(w, b_, a_):
            lo = jax.lax.dot_general(b_, a_, (((1,), (0,)), ((), ())),
                                     preferred_element_type=jnp.float32)
            return (w + lo).astype(jnp.bfloat16)

        wgu_s[0:_F, :] = merged(wg_ref[0], bg_ref[0], ag_ref[0])
        wgu_s[_F:2 * _F, :] = merged(wu_ref[0], bu_ref[0], au_ref[0])
        wdm_s[...] = merged(wd_ref[0], bd_ref[0], ad_ref[0])

    @pl.when(j < nact_ref[0])
    def _compute():
        xb = xs_ref[...].astype(jnp.bfloat16)
        gu = jax.lax.dot_general(xb, wgu_s[...], (((1,), (1,)), ((), ())),
                                 preferred_element_type=jnp.float32)
        g = gu[:, :_F]
        u = gu[:, _F:]
        h = (g * (1.0 / (1.0 + jnp.exp(-g))) * u).astype(jnp.bfloat16)
        ys_ref[...] = jax.lax.dot_general(h, wdm_s[...],
                                          (((1,), (1,)), ((), ())),
                                          preferred_element_type=jnp.float32)


def _sc_gather_body(ys_hbm, posa_hbm, posb_hbm, ya_hbm, yb_hbm,
                    idx_v, rows_v, sem):
    wid = lax.axis_index("s") * _NC + lax.axis_index("c")
    base = wid * _TPW
    pltpu.sync_copy(posa_hbm.at[pl.ds(base, _TPW)], idx_v)
    pltpu.async_copy(ys_hbm.at[idx_v], rows_v, sem).wait()
    pltpu.sync_copy(rows_v, ya_hbm.at[pl.ds(base, _TPW)])
    pltpu.sync_copy(posb_hbm.at[pl.ds(base, _TPW)], idx_v)
    pltpu.async_copy(ys_hbm.at[idx_v], rows_v, sem).wait()
    pltpu.sync_copy(rows_v, yb_hbm.at[pl.ds(base, _TPW)])


def _combine_body(ya_ref, yb_ref, wp_ref, out_ref):
    out_ref[...] = (wp_ref[:, 0:1] * ya_ref[...]
                    + wp_ref[:, 1:2] * yb_ref[...])


def _dispatch(x16, Wr, interpret=False):
    return pl.pallas_call(
        _dispatch_body,
        grid=(1,),
        in_specs=[pl.BlockSpec((_T, _D), lambda i: (0, 0)),
                  pl.BlockSpec((_E, _D), lambda i: (0, 0))],
        out_specs=[pl.BlockSpec((_T, 1), lambda i: (0, 0)),
                   pl.BlockSpec((_T, 1), lambda i: (0, 0)),
                   pl.BlockSpec((_T, 2), lambda i: (0, 0)),
                   pl.BlockSpec((128, 1), lambda i: (0, 0)),
                   pl.BlockSpec((1, 1), lambda i: (0, 0))],
        out_shape=[jax.ShapeDtypeStruct((_T, 1), jnp.int32),
                   jax.ShapeDtypeStruct((_T, 1), jnp.int32),
                   jax.ShapeDtypeStruct((_T, 2), jnp.float32),
                   jax.ShapeDtypeStruct((128, 1), jnp.int32),
                   jax.ShapeDtypeStruct((1, 1), jnp.int32)],
        interpret=interpret,
    )(x16, Wr)


def _grouped(te, nact, xs, Wg, Wu, Wd, Ag, Bg, Au, Bu, Ad, Bd,
             interpret=False):
    grid_spec = pltpu.PrefetchScalarGridSpec(
        num_scalar_prefetch=2,
        grid=(_NT,),
        in_specs=[
            pl.BlockSpec((_TILE, _D), lambda j, te_r, na_r: (j, 0)),
            pl.BlockSpec((1, _F, _D), lambda j, te_r, na_r: (te_r[j], 0, 0)),
            pl.BlockSpec((1, _F, _D), lambda j, te_r, na_r: (te_r[j], 0, 0)),
            pl.BlockSpec((1, _D, _F), lambda j, te_r, na_r: (te_r[j], 0, 0)),
            pl.BlockSpec((1, _R, _D), lambda j, te_r, na_r: (te_r[j], 0, 0)),
            pl.BlockSpec((1, _F, _R), lambda j, te_r, na_r: (te_r[j], 0, 0)),
            pl.BlockSpec((1, _R, _D), lambda j, te_r, na_r: (te_r[j], 0, 0)),
            pl.BlockSpec((1, _F, _R), lambda j, te_r, na_r: (te_r[j], 0, 0)),
            pl.BlockSpec((1, _R, _F), lambda j, te_r, na_r: (te_r[j], 0, 0)),
            pl.BlockSpec((1, _D, _R), lambda j, te_r, na_r: (te_r[j], 0, 0)),
        ],
        out_specs=pl.BlockSpec((_TILE, _D), lambda j, te_r, na_r: (j, 0)),
        scratch_shapes=[pltpu.VMEM((2 * _F, _D), jnp.bfloat16),
                        pltpu.VMEM((_D, _F), jnp.bfloat16)],
    )
    return pl.pallas_call(
        _grouped_body,
        grid_spec=grid_spec,
        out_shape=jax.ShapeDtypeStruct((_NSLOT, _D), jnp.float32),
        interpret=interpret,
    )(te, nact, xs, Wg, Wu, Wd, Ag, Bg, Au, Bu, Ad, Bd)


def _combine(ya, yb, wpair, interpret=False):
    return pl.pallas_call(
        _combine_body,
        grid=(4,),
        in_specs=[pl.BlockSpec((_T // 4, _D), lambda i: (i, 0)),
                  pl.BlockSpec((_T // 4, _D), lambda i: (i, 0)),
                  pl.BlockSpec((_T // 4, 2), lambda i: (i, 0))],
        out_specs=pl.BlockSpec((_T // 4, _D), lambda i: (i, 0)),
        out_shape=jax.ShapeDtypeStruct((_T, _D), jnp.float32),
        interpret=interpret,
    )(ya, yb, wpair)


@functools.lru_cache(maxsize=None)
def _sc_kernels():
    mesh = plsc.VectorSubcoreMesh(core_axis_name="c", subcore_axis_name="s")
    scatter = functools.partial(
        pl.kernel, mesh=mesh,
        out_type=jax.ShapeDtypeStruct((_NSLOT, _D), jnp.float32),
        scratch_types=[pltpu.VMEM((_TPW,), jnp.int32),
                       pltpu.VMEM((_TPW,), jnp.int32),
                       pltpu.VMEM((_TPW, _D), jnp.float32),
                       pltpu.SemaphoreType.DMA],
    )(_sc_scatter_body)
    gather = functools.partial(
        pl.kernel, mesh=mesh,
        out_type=(jax.ShapeDtypeStruct((_T, _D), jnp.float32),
                  jax.ShapeDtypeStruct((_T, _D), jnp.float32)),
        scratch_types=[pltpu.VMEM((_TPW,), jnp.int32),
                       pltpu.VMEM((_TPW, _D), jnp.float32),
                       pltpu.SemaphoreType.DMA],
    )(_sc_gather_body)
    return scatter, gather


@jax.jit
def kernel(hidden_states, Wr, Wg, Wu, Wd, Ag, Bg, Au, Bu, Ad, Bd):
    x = hidden_states.reshape(_T, _D)
    x16 = x.astype(jnp.bfloat16)

    posa, posb, wpair, te, nact = _dispatch(x16, Wr)
    posa1 = posa.reshape(_T)
    posb1 = posb.reshape(_T)

    sc_scatter, sc_gather = _sc_kernels()
    xs = sc_scatter(x, posa1, posb1)
    ys = _grouped(te.reshape(128), nact.reshape(1), xs,
                  Wg, Wu, Wd, Ag, Bg, Au, Bu, Ad, Bd)
    ya, yb = sc_gather(ys, posa1, posb1)
    y = _combine(ya, yb, wpair)
    return y.reshape(_B, _S, _D)


# speedup vs baseline: 1.4837x; 1.4837x over previous
import functools

import jax
import jax.numpy as jnp
from jax.experimental import pallas as pl
from jax.experimental.pallas import tpu as pltpu

_B, _S, _D = 1, 2048, 1024
_E, _K, _F, _R = 8, 2, 512, 8
_T = _B * _S
_TC = 512
_EF = _E * _F


def _moe_body(x16_ref, wr_ref, wg_ref, wu_ref, wd_ref, ag_ref, bg_ref,
              au_ref, bu_ref, ad_ref, bd_ref, out_ref,
              comb_ref, h_ref, wdall_ref, wgu_ref):
    e = pl.program_id(0)

    @pl.when(e == 0)
    def _router():
        logits = jax.lax.dot_general(
            x16_ref[...], wr_ref[...].astype(jnp.bfloat16),
            (((1,), (1,)), ((), ())),
            preferred_element_type=jnp.float32)
        m = jnp.max(logits, axis=-1, keepdims=True)
        p = jnp.exp(logits - m)
        p = p / jnp.sum(p, axis=-1, keepdims=True)
        lane = jax.lax.broadcasted_iota(jnp.int32, (_T, _E), 1)
        p1 = jnp.max(p, axis=-1, keepdims=True)
        i1 = jnp.min(jnp.where(p == p1, lane, _E), axis=-1, keepdims=True)
        m1 = lane == i1
        pr = jnp.where(m1, -1.0, p)
        p2 = jnp.max(pr, axis=-1, keepdims=True)
        i2 = jnp.min(jnp.where(pr == p2, lane, _E), axis=-1, keepdims=True)
        m2 = lane == i2
        comb_ref[...] = (jnp.where(m1, p, 0.0) + jnp.where(m2, p, 0.0)) / (p1 + p2)

    @pl.when(e < _E)
    def _expert():
        def merged_t(w, b_, a_):
            lo = jax.lax.dot_general(a_, b_, (((0,), (1,)), ((), ())),
                                     preferred_element_type=jnp.float32)
            return (w.T + lo).astype(jnp.bfloat16)

        col = pl.multiple_of(e * _F, _F)
        wgu_ref[:, 0:_F] = merged_t(wg_ref[0], bg_ref[0], ag_ref[0])
        wgu_ref[:, _F:2 * _F] = merged_t(wu_ref[0], bu_ref[0], au_ref[0])
        wdall_ref[pl.ds(col, _F), :] = merged_t(wd_ref[0], bd_ref[0], ad_ref[0])

        for c in range(_T // _TC):
            sl = pl.ds(c * _TC, _TC)
            gu = jax.lax.dot_general(x16_ref[sl, :], wgu_ref[...],
                                     (((1,), (0,)), ((), ())),
                                     preferred_element_type=jnp.float32)
            g = gu[:, :_F]
            u = gu[:, _F:]
            lane = jax.lax.broadcasted_iota(jnp.int32, (_TC, _E), 1)
            cw = jnp.sum(jnp.where(lane == e, comb_ref[sl, :], 0.0), axis=-1,
                         keepdims=True)
            h_ref[sl, pl.ds(col, _F)] = (
                g * (1.0 / (1.0 + jnp.exp(-g))) * u * cw).astype(jnp.bfloat16)

    @pl.when(e == _E)
    def _down():
        for c in range(_T // _TC):
            sl = pl.ds(c * _TC, _TC)
            out_ref[sl, :] = jax.lax.dot_general(
                h_ref[sl, :], wdall_ref[...], (((1,), (0,)), ((), ())),
                preferred_element_type=jnp.float32)


@functools.partial(jax.jit, static_argnames=("interpret",))
def kernel(hidden_states, Wr, Wg, Wu, Wd, Ag, Bg, Au, Bu, Ad, Bd,
           interpret=False):
    x16 = hidden_states.reshape(_T, _D).astype(jnp.bfloat16)

    def eb(e):
        ec = jnp.minimum(e, _E - 1)
        return ec

    y = pl.pallas_call(
        _moe_body,
        grid=(_E + 1,),
        in_specs=[
            pl.BlockSpec((_T, _D), lambda e: (0, 0)),
            pl.BlockSpec((_E, _D), lambda e: (0, 0)),
            pl.BlockSpec((1, _F, _D), lambda e: (eb(e), 0, 0)),
            pl.BlockSpec((1, _F, _D), lambda e: (eb(e), 0, 0)),
            pl.BlockSpec((1, _D, _F), lambda e: (eb(e), 0, 0)),
            pl.BlockSpec((1, _R, _D), lambda e: (eb(e), 0, 0)),
            pl.BlockSpec((1, _F, _R), lambda e: (eb(e), 0, 0)),
            pl.BlockSpec((1, _R, _D), lambda e: (eb(e), 0, 0)),
            pl.BlockSpec((1, _F, _R), lambda e: (eb(e), 0, 0)),
            pl.BlockSpec((1, _R, _F), lambda e: (eb(e), 0, 0)),
            pl.BlockSpec((1, _D, _R), lambda e: (eb(e), 0, 0)),
        ],
        out_specs=pl.BlockSpec((_T, _D), lambda e: (0, 0)),
        out_shape=jax.ShapeDtypeStruct((_T, _D), jnp.float32),
        scratch_shapes=[pltpu.VMEM((_T, _E), jnp.float32),
                        pltpu.VMEM((_T, _EF), jnp.bfloat16),
                        pltpu.VMEM((_EF, _D), jnp.bfloat16),
                        pltpu.VMEM((_D, 2 * _F), jnp.bfloat16)],
        interpret=interpret,
    )(x16, Wr, Wg, Wu, Wd, Ag, Bg, Au, Bu, Ad, Bd)

    return y.reshape(_B, _S, _D)


# speedup vs baseline: 1.6253x; 1.0954x over previous
import functools

import jax
import jax.numpy as jnp
from jax.experimental import pallas as pl
from jax.experimental.pallas import tpu as pltpu

_B, _S, _D = 1, 2048, 1024
_E, _K, _F, _R = 8, 2, 512, 8
_T = _B * _S
_TC = 256
_EF = _E * _F


def _moe_body(x16_ref, wr_ref, wg_ref, wu_ref, wd_ref, ag_ref, bg_ref,
              au_ref, bu_ref, ad_ref, bd_ref, out_ref,
              comb_ref, h_ref, wdall_ref, wgu_ref):
    s = pl.program_id(0)

    @pl.when(s == 0)
    def _router():
        logits = jax.lax.dot_general(
            x16_ref[...], wr_ref[...].astype(jnp.bfloat16),
            (((1,), (1,)), ((), ())),
            preferred_element_type=jnp.float32)
        m = jnp.max(logits, axis=-1, keepdims=True)
        p = jnp.exp(logits - m)
        p = p / jnp.sum(p, axis=-1, keepdims=True)
        lane = jax.lax.broadcasted_iota(jnp.int32, (_T, _E), 1)
        p1 = jnp.max(p, axis=-1, keepdims=True)
        i1 = jnp.min(jnp.where(p == p1, lane, _E), axis=-1, keepdims=True)
        m1 = lane == i1
        pr = jnp.where(m1, -1.0, p)
        p2 = jnp.max(pr, axis=-1, keepdims=True)
        i2 = jnp.min(jnp.where(pr == p2, lane, _E), axis=-1, keepdims=True)
        m2 = lane == i2
        comb_ref[...] = ((jnp.where(m1, p, 0.0) + jnp.where(m2, p, 0.0))
                         / (p1 + p2)).astype(jnp.bfloat16)

    @pl.when(s < _E)
    def _merge():
        def merged(w, bt_, a_):
            lo = jax.lax.dot_general(bt_, a_, (((0,), (0,)), ((), ())),
                                     preferred_element_type=jnp.float32)
            return (w + lo).astype(jnp.bfloat16)

        buf = jax.lax.rem(s, 2)
        wgu_ref[buf, 0:_F, :] = merged(wg_ref[0], bg_ref[0], ag_ref[0])
        wgu_ref[buf, _F:2 * _F, :] = merged(wu_ref[0], bu_ref[0], au_ref[0])
        col = pl.multiple_of(s * _F, _F)
        wdall_ref[:, pl.ds(col, _F)] = merged(wd_ref[0], bd_ref[0], ad_ref[0])

    @pl.when((s >= 1) & (s <= _E))
    def _expert():
        e = s - 1
        buf = jax.lax.rem(e, 2)
        colh = pl.multiple_of(e * _F, _F)
        for c in range(_T // _TC):
            sl = pl.ds(c * _TC, _TC)
            gu = jax.lax.dot_general(x16_ref[sl, :], wgu_ref[buf],
                                     (((1,), (1,)), ((), ())),
                                     preferred_element_type=jnp.float32)
            g = gu[:, :_F]
            u = gu[:, _F:]
            lane = jax.lax.broadcasted_iota(jnp.int32, (_TC, _E), 1)
            cw = jnp.sum(jnp.where(lane == e, comb_ref[sl, :].astype(jnp.float32),
                                   0.0), axis=-1, keepdims=True)
            h_ref[sl, pl.ds(colh, _F)] = (
                g * (1.0 / (1.0 + jnp.exp(-g))) * u * cw).astype(jnp.bfloat16)

    @pl.when(s == _E + 1)
    def _down():
        for c in range(_T // _TC):
            sl = pl.ds(c * _TC, _TC)
            out_ref[sl, :] = jax.lax.dot_general(
                h_ref[sl, :], wdall_ref[...], (((1,), (1,)), ((), ())),
                preferred_element_type=jnp.float32)


@functools.partial(jax.jit, static_argnames=("interpret",))
def kernel(hidden_states, Wr, Wg, Wu, Wd, Ag, Bg, Au, Bu, Ad, Bd,
           interpret=False):
    x16 = hidden_states.reshape(_T, _D).astype(jnp.bfloat16)
    BgT = jnp.swapaxes(Bg, 1, 2)
    BuT = jnp.swapaxes(Bu, 1, 2)
    BdT = jnp.swapaxes(Bd, 1, 2)

    def eb(s):
        return jnp.minimum(s, _E - 1)

    y = pl.pallas_call(
        _moe_body,
        grid=(_E + 2,),
        in_specs=[
            pl.BlockSpec((_T, _D), lambda s: (0, 0)),
            pl.BlockSpec((_E, _D), lambda s: (0, 0)),
            pl.BlockSpec((1, _F, _D), lambda s: (eb(s), 0, 0)),
            pl.BlockSpec((1, _F, _D), lambda s: (eb(s), 0, 0)),
            pl.BlockSpec((1, _D, _F), lambda s: (eb(s), 0, 0)),
            pl.BlockSpec((1, _R, _D), lambda s: (eb(s), 0, 0)),
            pl.BlockSpec((1, _R, _F), lambda s: (eb(s), 0, 0)),
            pl.BlockSpec((1, _R, _D), lambda s: (eb(s), 0, 0)),
            pl.BlockSpec((1, _R, _F), lambda s: (eb(s), 0, 0)),
            pl.BlockSpec((1, _R, _F), lambda s: (eb(s), 0, 0)),
            pl.BlockSpec((1, _R, _D), lambda s: (eb(s), 0, 0)),
        ],
        out_specs=pl.BlockSpec((_T, _D), lambda s: (0, 0)),
        out_shape=jax.ShapeDtypeStruct((_T, _D), jnp.float32),
        scratch_shapes=[pltpu.VMEM((_T, _E), jnp.bfloat16),
                        pltpu.VMEM((_T, _EF), jnp.bfloat16),
                        pltpu.VMEM((_D, _EF), jnp.bfloat16),
                        pltpu.VMEM((2, 2 * _F, _D), jnp.bfloat16)],
        interpret=interpret,
    )(x16, Wr, Wg, Wu, Wd, Ag, BgT, Au, BuT, Ad, BdT)

    return y.reshape(_B, _S, _D)
